# baseline (speedup 1.0000x reference)
import functools

import jax
import jax.numpy as jnp
from jax import lax
from jax.experimental import pallas as pl
from jax.experimental.pallas import tpu as pltpu
from jax.experimental.pallas import tpu_sc as plsc

N_EXPERTS = 8
TOP_K = 2
AUX_COEFF = 0.01
LANES = 128
NEG = -1e30

SC_CORES = 2
SC_SUBCORES = 16
NW = SC_CORES * SC_SUBCORES

TMS = 256


def _router_body(x_ref, wg_ref, tri_ref,
                 meta_i_ref, meta_f_ref, cnt_ref, aux_ref,
                 carry, psum, *, n_tokens):
    t = pl.program_id(0)
    nt = pl.num_programs(0)
    x = x_ref[...]
    logits = jnp.dot(x, wg_ref[...], preferred_element_type=jnp.float32)
    tm = logits.shape[0]
    col = lax.broadcasted_iota(jnp.int32, (tm, LANES), 1)
    valid = col < N_EXPERTS
    ml = jnp.where(valid, logits, NEG)
    m1 = jnp.max(ml, axis=1, keepdims=True)
    i1 = jnp.min(jnp.where(ml == m1, col, LANES), axis=1, keepdims=True)
    ml2 = jnp.where(col == i1, NEG, ml)
    m2 = jnp.max(ml2, axis=1, keepdims=True)
    i2 = jnp.min(jnp.where(ml2 == m2, col, LANES), axis=1, keepdims=True)
    r = jnp.exp(m2 - m1)
    s1 = 1.0 / (1.0 + r)
    s2 = r / (1.0 + r)
    oh1 = jnp.where(col == i1, 1.0, 0.0)
    oh2 = jnp.where(col == i2, 1.0, 0.0)
    oh = oh1 + oh2

    @pl.when(t == 0)
    def _init():
        carry[...] = jnp.zeros_like(carry)
        psum[...] = jnp.zeros_like(psum)

    incl = jnp.dot(tri_ref[...], oh, preferred_element_type=jnp.float32)
    incl = incl + carry[...]
    rank1 = jnp.sum(oh1 * (incl - 1.0), axis=1, keepdims=True)
    rank2 = jnp.sum(oh2 * (incl - 1.0), axis=1, keepdims=True)
    meta_i_ref[...] = (jnp.where(col == 0, i1, 0)
                       + jnp.where(col == 1, i2, 0)
                       + jnp.where(col == 2, rank1.astype(jnp.int32), 0)
                       + jnp.where(col == 3, rank2.astype(jnp.int32), 0))
    meta_f_ref[...] = (jnp.where(col == 0, s1, 0.0)
                       + jnp.where(col == 1, s2, 0.0))
    ex = jnp.where(valid, jnp.exp(ml - m1), 0.0)
    z = jnp.sum(ex, axis=1, keepdims=True)
    psum[...] += jnp.sum(ex / z, axis=0, keepdims=True)
    carry[...] += jnp.sum(oh, axis=0, keepdims=True)

    @pl.when(t == nt - 1)
    def _fin():
        cnt_ref[...] = carry[...]
        inv_n = 1.0 / float(n_tokens)
        aux_ref[...] = (AUX_COEFF * N_EXPERTS * inv_n * inv_n
                        * jnp.sum(carry[...] * psum[...], axis=1, keepdims=True))


def _router(x_flat, wg_pad, tri, tm):
    n, d = x_flat.shape
    nt = n // tm
    return pl.pallas_call(
        functools.partial(_router_body, n_tokens=n),
        grid=(nt,),
        in_specs=[
            pl.BlockSpec((tm, d), lambda t: (t, 0)),
            pl.BlockSpec((d, LANES), lambda t: (0, 0)),
            pl.BlockSpec((tm, tm), lambda t: (0, 0)),
        ],
        out_specs=[
            pl.BlockSpec((tm, LANES), lambda t: (t, 0)),
            pl.BlockSpec((tm, LANES), lambda t: (t, 0)),
            pl.BlockSpec((1, LANES), lambda t: (0, 0)),
            pl.BlockSpec((1, 1), lambda t: (0, 0)),
        ],
        out_shape=[
            jax.ShapeDtypeStruct((n, LANES), jnp.int32),
            jax.ShapeDtypeStruct((n, LANES), jnp.float32),
            jax.ShapeDtypeStruct((1, LANES), jnp.float32),
            jax.ShapeDtypeStruct((1, 1), jnp.float32),
        ],
        scratch_shapes=[pltpu.VMEM((1, LANES), jnp.float32),
                        pltpu.VMEM((1, LANES), jnp.float32)],
        compiler_params=pltpu.CompilerParams(
            dimension_semantics=("arbitrary",)),
    )(x_flat, wg_pad, tri)


def _dispatch_gather(x_flat, e2, r2, w2, base_m, s_pad):
    n, d = x_flat.shape
    ch = n // NW
    ng = ch // 16
    mesh = plsc.VectorSubcoreMesh(core_axis_name="c", subcore_axis_name="s")

    @functools.partial(
        pl.kernel, mesh=mesh,
        out_type=[jax.ShapeDtypeStruct((2, n), jnp.int32),
                  jax.ShapeDtypeStruct((s_pad,), jnp.float32),
                  jax.ShapeDtypeStruct((s_pad, d), jnp.float32)],
        scratch_types=[
            pltpu.VMEM((N_EXPERTS, 16), jnp.int32),
            pltpu.VMEM((ch,), jnp.int32),
            pltpu.VMEM((ch,), jnp.int32),
            pltpu.VMEM((ch,), jnp.float32),
            pltpu.VMEM((ch,), jnp.int32),
            pltpu.VMEM((2, ng, 16), jnp.int32),
            pltpu.VMEM((16, d), jnp.float32),
            pltpu.VMEM((16, d), jnp.float32),
            pltpu.SemaphoreType.DMA,
        ],
    )
    def k(x_hbm, e2_hbm, r2_hbm, w2_hbm, base_hbm,
          pos2_hbm, ws_hbm, xs_hbm,
          base_v, e_v, r_v, w_v, p_v, pm, xbufa, xbufb, sem):
        wid = lax.axis_index("s") * SC_CORES + lax.axis_index("c")
        off = wid * ch
        pltpu.sync_copy(base_hbm, base_v)
        bvec = [base_v[e, :] for e in range(N_EXPERTS)]
        for kslot in range(TOP_K):
            pltpu.sync_copy(e2_hbm.at[kslot, pl.ds(off, ch)], e_v)
            pltpu.sync_copy(r2_hbm.at[kslot, pl.ds(off, ch)], r_v)
            pltpu.sync_copy(w2_hbm.at[kslot, pl.ds(off, ch)], w_v)
            for i in range(ng):
                sl = pl.ds(i * 16, 16)
                ev = e_v[sl]
                p = r_v[sl]
                for e in range(N_EXPERTS):
                    p = p + jnp.where(ev == e, bvec[e], 0)
                p_v[sl] = p
                pm[kslot, i, :] = p
            pltpu.sync_copy(p_v, pos2_hbm.at[kslot, pl.ds(off, ch)])
            pltpu.sync_copy(w_v, ws_hbm.at[p_v])
        bufs = (xbufa, xbufb)
        descs = [None] * ng
        for g in range(ng):
            buf = bufs[g % 2]
            if g >= 2:
                descs[g - 2][0].wait()
                descs[g - 2][1].wait()
            pltpu.sync_copy(x_hbm.at[pl.ds(off + g * 16, 16)], buf)
            d0 = pltpu.async_copy(buf, xs_hbm.at[pm.at[0, g]], sem)
            d1 = pltpu.async_copy(buf, xs_hbm.at[pm.at[1, g]], sem)
            descs[g] = (d0, d1)
        for g in (ng - 2, ng - 1):
            descs[g][0].wait()
            descs[g][1].wait()

    return k(x_flat, e2, r2, w2, base_m)


def _gmm_body(te_ref, xs_ref, ws_ref, w1_ref, w3_ref, w2_ref, ys_ref):
    x = xs_ref[...]
    h = jnp.dot(x, w1_ref[0], preferred_element_type=jnp.float32)
    u = jnp.dot(x, w3_ref[0], preferred_element_type=jnp.float32)
    act = (h * jax.nn.sigmoid(h)) * u
    y = jnp.dot(act, w2_ref[0], preferred_element_type=jnp.float32)
    ys_ref[...] = y * ws_ref[...]


def _gmm(xs, ws2d, W1, W3, W2, te, s_pad):
    d = xs.shape[1]
    f = W1.shape[2]
    t_pad = s_pad // TMS
    grid_spec = pltpu.PrefetchScalarGridSpec(
        num_scalar_prefetch=1,
        grid=(t_pad,),
        in_specs=[
            pl.BlockSpec((TMS, d), lambda t, te: (t, 0)),
            pl.BlockSpec((TMS, 1), lambda t, te: (t, 0)),
            pl.BlockSpec((1, d, f), lambda t, te: (te[t], 0, 0)),
            pl.BlockSpec((1, d, f), lambda t, te: (te[t], 0, 0)),
            pl.BlockSpec((1, f, d), lambda t, te: (te[t], 0, 0)),
        ],
        out_specs=pl.BlockSpec((TMS, d), lambda t, te: (t, 0)),
    )
    return pl.pallas_call(
        _gmm_body,
        grid_spec=grid_spec,
        out_shape=jax.ShapeDtypeStruct((s_pad, d), jnp.float32),
        compiler_params=pltpu.CompilerParams(
            dimension_semantics=("arbitrary",),
            vmem_limit_bytes=100 * 1024 * 1024),
    )(te, xs, ws2d, W1, W3, W2)


def _combine(ys, posI, n, d):
    gt = 8
    ch = n // NW
    ng = ch // gt
    mesh = plsc.VectorSubcoreMesh(core_axis_name="c", subcore_axis_name="s")

    @functools.partial(
        pl.kernel, mesh=mesh,
        out_type=jax.ShapeDtypeStruct((n, d), jnp.float32),
        scratch_types=[
            pltpu.VMEM((2 * gt, d), jnp.float32),
            pltpu.VMEM((2 * gt, d), jnp.float32),
            pltpu.VMEM((gt, d), jnp.float32),
            pltpu.VMEM((gt, d), jnp.float32),
            pltpu.VMEM((ng, 2 * gt), jnp.int32),
            pltpu.SemaphoreType.DMA,
            pltpu.SemaphoreType.DMA,
        ],
    )
    def k(ys_hbm, posI_hbm, out_hbm, gbufa, gbufb, obufa, obufb, pim,
          gsem, wsem):
        wid = lax.axis_index("s") * SC_CORES + lax.axis_index("c")
        toff = wid * ch
        pltpu.sync_copy(posI_hbm.at[pl.ds(wid * ng, ng)], pim)
        gbufs = (gbufa, gbufb)
        obufs = (obufa, obufb)
        gd = [None] * ng
        wd = [None] * ng
        gd[0] = pltpu.async_copy(ys_hbm.at[pim.at[0]], gbufa, gsem)
        for g in range(ng):
            gd[g].wait()
            if g + 1 < ng:
                gd[g + 1] = pltpu.async_copy(
                    ys_hbm.at[pim.at[g + 1]], gbufs[(g + 1) % 2], gsem)
            gbuf = gbufs[g % 2]
            obuf = obufs[g % 2]
            if g >= 2:
                wd[g - 2].wait()

            def add_body(c, _, gbuf=gbuf, obuf=obuf):
                sl = pl.ds(c * 16, 16)
                for pair in range(gt):
                    obuf[pair, sl] = gbuf[2 * pair, sl] + gbuf[2 * pair + 1, sl]
                return 0

            lax.fori_loop(0, d // 16, add_body, 0)
            wd[g] = pltpu.async_copy(obuf, out_hbm.at[pl.ds(toff + g * gt, gt)],
                                     wsem)
        wd[ng - 2].wait()
        wd[ng - 1].wait()

    return k(ys, posI)


def kernel(x, Wg, W1, W3, W2):
    b, t, d = x.shape
    n = b * t
    f = W1.shape[2]
    s_pad = TOP_K * n + N_EXPERTS * TMS
    x_flat = x.reshape(n, d)
    wg_pad = jnp.pad(Wg, ((0, 0), (0, LANES - N_EXPERTS)))
    tm = min(512, n)
    tri = jnp.tril(jnp.ones((tm, tm), jnp.float32))

    meta_i, meta_f, cnts, aux = _router(x_flat, wg_pad, tri, tm)

    e2 = jnp.stack([meta_i[:, 0], meta_i[:, 1]])
    r2 = jnp.stack([meta_i[:, 2], meta_i[:, 3]])
    w2 = jnp.stack([meta_f[:, 0], meta_f[:, 1]])
    counts = cnts[0, :N_EXPERTS].astype(jnp.int32)
    cnt_pad = ((counts + TMS - 1) // TMS) * TMS
    ends = jnp.cumsum(cnt_pad)
    base = (ends - cnt_pad).astype(jnp.int32)
    t_pad = s_pad // TMS
    tile_starts = jnp.arange(t_pad, dtype=jnp.int32) * TMS
    te = jnp.sum((tile_starts[:, None] >= ends[None, :]).astype(jnp.int32),
                 axis=1)
    te = jnp.minimum(te, N_EXPERTS - 1).astype(jnp.int32)

    base_m = jnp.broadcast_to(base[:, None], (N_EXPERTS, 16))
    pos2, ws, xs = _dispatch_gather(x_flat, e2, r2, w2, base_m, s_pad)
    ys = _gmm(xs, ws.reshape(s_pad, 1), W1, W3, W2, te, s_pad)
    posI = jnp.stack([pos2[0], pos2[1]], axis=1).reshape(n // 8, 16)
    out = _combine(ys, posI, n, d)
    return out.reshape(b, t, d), aux[0, 0]


# speedup vs baseline: 1.0105x; 1.0105x over previous
import functools

import jax
import jax.numpy as jnp
from jax import lax
from jax.experimental import pallas as pl
from jax.experimental.pallas import tpu as pltpu
from jax.experimental.pallas import tpu_sc as plsc

N_EXPERTS = 8
TOP_K = 2
AUX_COEFF = 0.01
LANES = 128
NEG = -1e30

SC_CORES = 2
SC_SUBCORES = 16
NW = SC_CORES * SC_SUBCORES

TMS = 128


def _router_body(x_ref, wg_ref, tri_ref,
                 meta_i_ref, meta_f_ref, cnt_ref, aux_ref,
                 carry, psum, *, n_tokens):
    t = pl.program_id(0)
    nt = pl.num_programs(0)
    x = x_ref[...]
    logits = jnp.dot(x, wg_ref[...], preferred_element_type=jnp.float32)
    tm = logits.shape[0]
    col = lax.broadcasted_iota(jnp.int32, (tm, LANES), 1)
    valid = col < N_EXPERTS
    ml = jnp.where(valid, logits, NEG)
    m1 = jnp.max(ml, axis=1, keepdims=True)
    i1 = jnp.min(jnp.where(ml == m1, col, LANES), axis=1, keepdims=True)
    ml2 = jnp.where(col == i1, NEG, ml)
    m2 = jnp.max(ml2, axis=1, keepdims=True)
    i2 = jnp.min(jnp.where(ml2 == m2, col, LANES), axis=1, keepdims=True)
    r = jnp.exp(m2 - m1)
    s1 = 1.0 / (1.0 + r)
    s2 = r / (1.0 + r)
    oh1 = jnp.where(col == i1, 1.0, 0.0)
    oh2 = jnp.where(col == i2, 1.0, 0.0)
    oh = oh1 + oh2

    @pl.when(t == 0)
    def _init():
        carry[...] = jnp.zeros_like(carry)
        psum[...] = jnp.zeros_like(psum)

    incl = jnp.dot(tri_ref[...], oh, preferred_element_type=jnp.float32)
    incl = incl + carry[...]
    rank1 = jnp.sum(oh1 * (incl - 1.0), axis=1, keepdims=True)
    rank2 = jnp.sum(oh2 * (incl - 1.0), axis=1, keepdims=True)
    meta_i_ref[...] = (jnp.where(col == 0, i1, 0)
                       + jnp.where(col == 1, i2, 0)
                       + jnp.where(col == 2, rank1.astype(jnp.int32), 0)
                       + jnp.where(col == 3, rank2.astype(jnp.int32), 0))
    meta_f_ref[...] = (jnp.where(col == 0, s1, 0.0)
                       + jnp.where(col == 1, s2, 0.0))
    ex = jnp.where(valid, jnp.exp(ml - m1), 0.0)
    z = jnp.sum(ex, axis=1, keepdims=True)
    psum[...] += jnp.sum(ex / z, axis=0, keepdims=True)
    carry[...] += jnp.sum(oh, axis=0, keepdims=True)

    @pl.when(t == nt - 1)
    def _fin():
        cnt_ref[...] = carry[...]
        inv_n = 1.0 / float(n_tokens)
        aux_ref[...] = (AUX_COEFF * N_EXPERTS * inv_n * inv_n
                        * jnp.sum(carry[...] * psum[...], axis=1, keepdims=True))


def _router(x_flat, wg_pad, tri, tm):
    n, d = x_flat.shape
    nt = n // tm
    return pl.pallas_call(
        functools.partial(_router_body, n_tokens=n),
        grid=(nt,),
        in_specs=[
            pl.BlockSpec((tm, d), lambda t: (t, 0)),
            pl.BlockSpec((d, LANES), lambda t: (0, 0)),
            pl.BlockSpec((tm, tm), lambda t: (0, 0)),
        ],
        out_specs=[
            pl.BlockSpec((tm, LANES), lambda t: (t, 0)),
            pl.BlockSpec((tm, LANES), lambda t: (t, 0)),
            pl.BlockSpec((1, LANES), lambda t: (0, 0)),
            pl.BlockSpec((1, 1), lambda t: (0, 0)),
        ],
        out_shape=[
            jax.ShapeDtypeStruct((n, LANES), jnp.int32),
            jax.ShapeDtypeStruct((n, LANES), jnp.float32),
            jax.ShapeDtypeStruct((1, LANES), jnp.float32),
            jax.ShapeDtypeStruct((1, 1), jnp.float32),
        ],
        scratch_shapes=[pltpu.VMEM((1, LANES), jnp.float32),
                        pltpu.VMEM((1, LANES), jnp.float32)],
        compiler_params=pltpu.CompilerParams(
            dimension_semantics=("arbitrary",)),
    )(x_flat, wg_pad, tri)


def _dispatch_gather(x_flat, e2, r2, w2, base_m, s_pad):
    n, d = x_flat.shape
    ch = n // NW
    ng = ch // 16
    mesh = plsc.VectorSubcoreMesh(core_axis_name="c", subcore_axis_name="s")

    @functools.partial(
        pl.kernel, mesh=mesh,
        out_type=[jax.ShapeDtypeStruct((2, n), jnp.int32),
                  jax.ShapeDtypeStruct((s_pad,), jnp.float32),
                  jax.ShapeDtypeStruct((s_pad, d), jnp.float32)],
        scratch_types=[
            pltpu.VMEM((N_EXPERTS, 16), jnp.int32),
            pltpu.VMEM((ch,), jnp.int32),
            pltpu.VMEM((ch,), jnp.int32),
            pltpu.VMEM((ch,), jnp.float32),
            pltpu.VMEM((ch,), jnp.int32),
            pltpu.VMEM((2, ng, 16), jnp.int32),
            pltpu.VMEM((16, d), jnp.float32),
            pltpu.VMEM((16, d), jnp.float32),
            pltpu.VMEM((16, d), jnp.float32),
            pltpu.SemaphoreType.DMA,
            pltpu.SemaphoreType.DMA,
        ],
    )
    def k(x_hbm, e2_hbm, r2_hbm, w2_hbm, base_hbm,
          pos2_hbm, ws_hbm, xs_hbm,
          base_v, e_v, r_v, w_v, p_v, pm, xbufa, xbufb, xbufc, sem, lsem):
        wid = lax.axis_index("s") * SC_CORES + lax.axis_index("c")
        off = wid * ch
        pltpu.sync_copy(base_hbm, base_v)
        bvec = [base_v[e, :] for e in range(N_EXPERTS)]
        for kslot in range(TOP_K):
            pltpu.sync_copy(e2_hbm.at[kslot, pl.ds(off, ch)], e_v)
            pltpu.sync_copy(r2_hbm.at[kslot, pl.ds(off, ch)], r_v)
            pltpu.sync_copy(w2_hbm.at[kslot, pl.ds(off, ch)], w_v)
            for i in range(ng):
                sl = pl.ds(i * 16, 16)
                ev = e_v[sl]
                p = r_v[sl]
                for e in range(N_EXPERTS):
                    p = p + jnp.where(ev == e, bvec[e], 0)
                p_v[sl] = p
                pm[kslot, i, :] = p
            pltpu.sync_copy(p_v, pos2_hbm.at[kslot, pl.ds(off, ch)])
            pltpu.sync_copy(w_v, ws_hbm.at[p_v])
        bufs = (xbufa, xbufb, xbufc)
        descs = [None] * ng
        ld = [None] * ng
        for g in range(min(2, ng)):
            ld[g] = pltpu.async_copy(
                x_hbm.at[pl.ds(off + g * 16, 16)], bufs[g % 3], lsem)
        for g in range(ng):
            ld[g].wait()
            if g + 2 < ng:
                if g >= 1:
                    descs[g - 1][0].wait()
                    descs[g - 1][1].wait()
                ld[g + 2] = pltpu.async_copy(
                    x_hbm.at[pl.ds(off + (g + 2) * 16, 16)],
                    bufs[(g + 2) % 3], lsem)
            d0 = pltpu.async_copy(bufs[g % 3], xs_hbm.at[pm.at[0, g]], sem)
            d1 = pltpu.async_copy(bufs[g % 3], xs_hbm.at[pm.at[1, g]], sem)
            descs[g] = (d0, d1)
        for g in (ng - 3, ng - 2, ng - 1):
            descs[g][0].wait()
            descs[g][1].wait()

    return k(x_flat, e2, r2, w2, base_m)


def _gmm_body(te_ref, xs_ref, ws_ref, w1_ref, w3_ref, w2_ref, ys_ref):
    x = xs_ref[...]
    h = jnp.dot(x, w1_ref[0], preferred_element_type=jnp.float32)
    u = jnp.dot(x, w3_ref[0], preferred_element_type=jnp.float32)
    act = (h * jax.nn.sigmoid(h)) * u
    y = jnp.dot(act, w2_ref[0], preferred_element_type=jnp.float32)
    ys_ref[...] = y * ws_ref[...]


def _gmm(xs, ws2d, W1, W3, W2, te, s_pad):
    d = xs.shape[1]
    f = W1.shape[2]
    t_pad = s_pad // TMS
    grid_spec = pltpu.PrefetchScalarGridSpec(
        num_scalar_prefetch=1,
        grid=(t_pad,),
        in_specs=[
            pl.BlockSpec((TMS, d), lambda t, te: (t, 0)),
            pl.BlockSpec((TMS, 1), lambda t, te: (t, 0)),
            pl.BlockSpec((1, d, f), lambda t, te: (te[t], 0, 0)),
            pl.BlockSpec((1, d, f), lambda t, te: (te[t], 0, 0)),
            pl.BlockSpec((1, f, d), lambda t, te: (te[t], 0, 0)),
        ],
        out_specs=pl.BlockSpec((TMS, d), lambda t, te: (t, 0)),
    )
    return pl.pallas_call(
        _gmm_body,
        grid_spec=grid_spec,
        out_shape=jax.ShapeDtypeStruct((s_pad, d), jnp.float32),
        compiler_params=pltpu.CompilerParams(
            dimension_semantics=("arbitrary",),
            vmem_limit_bytes=100 * 1024 * 1024),
    )(te, xs, ws2d, W1, W3, W2)


def _combine(ys, posI, n, d):
    gt = 8
    ch = n // NW
    ng = ch // gt
    mesh = plsc.VectorSubcoreMesh(core_axis_name="c", subcore_axis_name="s")

    @functools.partial(
        pl.kernel, mesh=mesh,
        out_type=jax.ShapeDtypeStruct((n, d), jnp.float32),
        scratch_types=[
            pltpu.VMEM((2 * gt, d), jnp.float32),
            pltpu.VMEM((2 * gt, d), jnp.float32),
            pltpu.VMEM((gt, d), jnp.float32),
            pltpu.VMEM((gt, d), jnp.float32),
            pltpu.VMEM((ng, 2 * gt), jnp.int32),
            pltpu.SemaphoreType.DMA,
            pltpu.SemaphoreType.DMA,
        ],
    )
    def k(ys_hbm, posI_hbm, out_hbm, gbufa, gbufb, obufa, obufb, pim,
          gsem, wsem):
        wid = lax.axis_index("s") * SC_CORES + lax.axis_index("c")
        toff = wid * ch
        pltpu.sync_copy(posI_hbm.at[pl.ds(wid * ng, ng)], pim)
        gbufs = (gbufa, gbufb)
        obufs = (obufa, obufb)
        gd = [None] * ng
        wd = [None] * ng
        gd[0] = pltpu.async_copy(ys_hbm.at[pim.at[0]], gbufa, gsem)
        for g in range(ng):
            gd[g].wait()
            if g + 1 < ng:
                gd[g + 1] = pltpu.async_copy(
                    ys_hbm.at[pim.at[g + 1]], gbufs[(g + 1) % 2], gsem)
            gbuf = gbufs[g % 2]
            obuf = obufs[g % 2]
            if g >= 2:
                wd[g - 2].wait()

            def add_body(c, _, gbuf=gbuf, obuf=obuf):
                sl = pl.ds(c * 16, 16)
                for pair in range(gt):
                    obuf[pair, sl] = gbuf[2 * pair, sl] + gbuf[2 * pair + 1, sl]
                return 0

            lax.fori_loop(0, d // 16, add_body, 0)
            wd[g] = pltpu.async_copy(obuf, out_hbm.at[pl.ds(toff + g * gt, gt)],
                                     wsem)
        wd[ng - 2].wait()
        wd[ng - 1].wait()

    return k(ys, posI)


def kernel(x, Wg, W1, W3, W2):
    b, t, d = x.shape
    n = b * t
    f = W1.shape[2]
    s_pad = TOP_K * n + N_EXPERTS * TMS
    x_flat = x.reshape(n, d)
    wg_pad = jnp.pad(Wg, ((0, 0), (0, LANES - N_EXPERTS)))
    tm = min(512, n)
    tri = jnp.tril(jnp.ones((tm, tm), jnp.float32))

    meta_i, meta_f, cnts, aux = _router(x_flat, wg_pad, tri, tm)

    e2 = jnp.stack([meta_i[:, 0], meta_i[:, 1]])
    r2 = jnp.stack([meta_i[:, 2], meta_i[:, 3]])
    w2 = jnp.stack([meta_f[:, 0], meta_f[:, 1]])
    counts = cnts[0, :N_EXPERTS].astype(jnp.int32)
    cnt_pad = ((counts + TMS - 1) // TMS) * TMS
    ends = jnp.cumsum(cnt_pad)
    base = (ends - cnt_pad).astype(jnp.int32)
    t_pad = s_pad // TMS
    tile_starts = jnp.arange(t_pad, dtype=jnp.int32) * TMS
    te = jnp.sum((tile_starts[:, None] >= ends[None, :]).astype(jnp.int32),
                 axis=1)
    te = jnp.minimum(te, N_EXPERTS - 1).astype(jnp.int32)

    base_m = jnp.broadcast_to(base[:, None], (N_EXPERTS, 16))
    pos2, ws, xs = _dispatch_gather(x_flat, e2, r2, w2, base_m, s_pad)
    ys = _gmm(xs, ws.reshape(s_pad, 1), W1, W3, W2, te, s_pad)
    posI = jnp.stack([pos2[0], pos2[1]], axis=1).reshape(n // 8, 16)
    out = _combine(ys, posI, n, d)
    return out.reshape(b, t, d), aux[0, 0]
